# baseline (speedup 1.0000x reference)
import functools

import jax
import jax.numpy as jnp
from jax import lax
from jax.experimental import pallas as pl
from jax.experimental.pallas import tpu as pltpu
from jax.experimental.pallas import tpu_sc as plsc

F32 = jnp.float32
I32 = jnp.int32
HI = lax.Precision.HIGHEST
EPS = 1e-5


def _fps_body(xyz_ref, o3_ref, oT_ref, *, npoint):
    X = xyz_ref[...]
    Bv, _, Nv = X.shape
    x0 = X[:, 0, :]
    x1 = X[:, 1, :]
    x2 = X[:, 2, :]
    lane = lax.broadcasted_iota(I32, (Bv, Nv), 1)
    rec_lane = lax.broadcasted_iota(I32, (Bv, npoint), 1)

    def body(i, c):
        dist, far, nx, ny, nz = c
        m = lane == far
        cx = jnp.sum(jnp.where(m, x0, 0.0), 1, keepdims=True)
        cy = jnp.sum(jnp.where(m, x1, 0.0), 1, keepdims=True)
        cz = jnp.sum(jnp.where(m, x2, 0.0), 1, keepdims=True)
        rec = rec_lane == i
        nx = jnp.where(rec, cx, nx)
        ny = jnp.where(rec, cy, ny)
        nz = jnp.where(rec, cz, nz)
        d = (x0 - cx) ** 2 + (x1 - cy) ** 2 + (x2 - cz) ** 2
        dist = jnp.minimum(dist, d)
        far = jnp.argmax(dist, axis=1, keepdims=True).astype(I32)
        return dist, far, nx, ny, nz

    init = (
        jnp.full((Bv, Nv), 1e10, F32),
        jnp.zeros((Bv, 1), I32),
        jnp.zeros((Bv, npoint), F32),
        jnp.zeros((Bv, npoint), F32),
        jnp.zeros((Bv, npoint), F32),
    )
    _, _, nx, ny, nz = lax.fori_loop(0, npoint, body, init)
    o3_ref[:, 0, :] = nx
    o3_ref[:, 1, :] = ny
    o3_ref[:, 2, :] = nz
    oT_ref[:, :, 0] = nx
    oT_ref[:, :, 1] = ny
    oT_ref[:, :, 2] = nz


def _fps(xyz3n, npoint):
    B = xyz3n.shape[0]
    return pl.pallas_call(
        functools.partial(_fps_body, npoint=npoint),
        out_shape=(
            jax.ShapeDtypeStruct((B, 3, npoint), F32),
            jax.ShapeDtypeStruct((B, npoint, 3), F32),
        ),
    )(xyz3n)


def _sc_gather(table, idx, chunk=2048):
    info = plsc.get_sparse_core_info()
    NW = info.num_cores * info.num_subcores
    R = idx.shape[0]
    D = table.shape[1]
    b_per_w = R // NW
    nch = b_per_w // chunk
    mesh = plsc.VectorSubcoreMesh(core_axis_name="c", subcore_axis_name="s")

    @functools.partial(
        pl.kernel,
        mesh=mesh,
        compiler_params=pltpu.CompilerParams(use_tc_tiling_on_sc=False),
        out_type=jax.ShapeDtypeStruct((R, D), jnp.float32),
        scratch_types=[
            pltpu.VMEM((chunk,), jnp.int32),
            pltpu.VMEM((chunk, D), jnp.float32),
            pltpu.SemaphoreType.DMA,
        ],
    )
    def k(table_hbm, idx_hbm, out_hbm, idx_v, rows_v, sem):
        wid = lax.axis_index("s") * info.num_cores + lax.axis_index("c")
        base = wid * b_per_w
        for j in range(nch):
            off = base + j * chunk
            pltpu.sync_copy(idx_hbm.at[pl.ds(off, chunk)], idx_v)
            pltpu.async_copy(table_hbm.at[idx_v], rows_v, sem).wait()
            pltpu.sync_copy(rows_v, out_hbm.at[pl.ds(off, chunk)])

    return k(table, idx)


def _knn_body(p_ref, q_ref, idx_ref, *, k, offset):
    p = p_ref[0]
    q = q_ref[0]
    NP = q.shape[0]
    N = p.shape[1]
    px = p[0:1, :]
    py = p[1:2, :]
    pz = p[2:3, :]
    qx = q[:, 0:1]
    qy = q[:, 1:2]
    qz = q[:, 2:3]
    sqq = (qx * qx + qy * qy) + qz * qz
    sqp = (px * px + py * py) + pz * pz
    P = jnp.dot(q, p, preferred_element_type=F32,
                precision=lax.Precision.DEFAULT)
    D = (sqq + sqp) - 2.0 * P
    lane = lax.broadcasted_iota(I32, (NP, N), 1)
    kl = lax.broadcasted_iota(I32, (NP, k), 1)

    def body(j, c):
        D, acc = c
        sel = jnp.argmin(D, axis=1, keepdims=True).astype(I32)
        acc = jnp.where(kl == j, sel, acc)
        D = jnp.where(lane == sel, jnp.inf, D)
        return D, acc

    _, acc = lax.fori_loop(0, k, body, (D, jnp.zeros((NP, k), I32)))
    if offset:
        acc = acc + pl.program_id(0) * N
    idx_ref[0] = acc


def _knn(p3n, qT, k, offset=False):
    B, _, N = p3n.shape
    NP = qT.shape[1]
    return pl.pallas_call(
        functools.partial(_knn_body, k=k, offset=offset),
        grid=(B,),
        in_specs=[
            pl.BlockSpec((1, 3, N), lambda i: (i, 0, 0)),
            pl.BlockSpec((1, NP, 3), lambda i: (i, 0, 0)),
        ],
        out_specs=pl.BlockSpec((1, NP, k), lambda i: (i, 0, 0)),
        out_shape=jax.ShapeDtypeStruct((B, NP, k), I32),
    )(p3n, qT)


def _g1post_body(g_ref, q_ref, w_ref, b_ref, y_ref, s_ref, ss_ref, *, ksamp):
    g = g_ref[0]
    R, Dp = g.shape
    q = q_ref[0]
    nq = q.shape[0]
    qb = jnp.broadcast_to(q[:, None, :], (nq, ksamp, 3)).reshape(R, 3)
    x0 = g - jnp.concatenate([qb, jnp.zeros((R, Dp - 3), F32)], axis=1)
    y = jnp.dot(x0, w_ref[...], preferred_element_type=F32,
                precision=lax.Precision.DEFAULT) + b_ref[...]
    y_ref[0] = y

    @pl.when((pl.program_id(0) == 0) & (pl.program_id(1) == 0))
    def _():
        s_ref[...] = jnp.zeros_like(s_ref)
        ss_ref[...] = jnp.zeros_like(ss_ref)

    s_ref[...] += jnp.sum(y, 0, keepdims=True)
    ss_ref[...] += jnp.sum(y * y, 0, keepdims=True)


def _g1post(g, qT, Wp, b, ksamp, rows_per_chunk):
    B, RT, Dp = g.shape
    C = Wp.shape[1]
    nch = RT // rows_per_chunk
    qch = rows_per_chunk // ksamp
    return pl.pallas_call(
        functools.partial(_g1post_body, ksamp=ksamp),
        grid=(B, nch),
        in_specs=[
            pl.BlockSpec((1, rows_per_chunk, Dp), lambda i, j: (i, j, 0)),
            pl.BlockSpec((1, qch, 3), lambda i, j: (i, j, 0)),
            pl.BlockSpec((Dp, C), lambda i, j: (0, 0)),
            pl.BlockSpec((1, C), lambda i, j: (0, 0)),
        ],
        out_specs=(
            pl.BlockSpec((1, rows_per_chunk, C), lambda i, j: (i, j, 0)),
            pl.BlockSpec((1, C), lambda i, j: (0, 0)),
            pl.BlockSpec((1, C), lambda i, j: (0, 0)),
        ),
        out_shape=(
            jax.ShapeDtypeStruct((B, RT, C), F32),
            jax.ShapeDtypeStruct((1, C), F32),
            jax.ShapeDtypeStruct((1, C), F32),
        ),
    )(g, qT, Wp, b)


def _g2_body(idx_ref, tab_ref, q_ref, w_ref, b_ref, y_ref, s_ref, ss_ref, *, ksamp):
    idxc = idx_ref[0]
    R = idxc.shape[0]
    N = tab_ref.shape[1]
    S = (idxc == lax.broadcasted_iota(I32, (R, N), 1)).astype(F32)
    g = jnp.dot(S, tab_ref[0], preferred_element_type=F32, precision=HI)
    Cin = g.shape[1]
    q = q_ref[0]
    nq = q.shape[0]
    qb = jnp.broadcast_to(q[:, None, :], (nq, ksamp, 3)).reshape(R, 3)
    x = g - jnp.concatenate([qb, jnp.zeros((R, Cin - 3), F32)], axis=1)
    y = jnp.dot(x, w_ref[...], preferred_element_type=F32,
                precision=lax.Precision.DEFAULT) + b_ref[...]
    y_ref[0] = y

    @pl.when((pl.program_id(0) == 0) & (pl.program_id(1) == 0))
    def _():
        s_ref[...] = jnp.zeros_like(s_ref)
        ss_ref[...] = jnp.zeros_like(ss_ref)

    s_ref[...] += jnp.sum(y, 0, keepdims=True)
    ss_ref[...] += jnp.sum(y * y, 0, keepdims=True)


def _gather2_mlp(idxf, tab, qT, W, b, ksamp, rows_per_chunk):
    B, RT, _ = idxf.shape
    N, Cin = tab.shape[1], tab.shape[2]
    C = W.shape[1]
    nch = RT // rows_per_chunk
    qch = rows_per_chunk // ksamp
    return pl.pallas_call(
        functools.partial(_g2_body, ksamp=ksamp),
        grid=(B, nch),
        in_specs=[
            pl.BlockSpec((1, rows_per_chunk, 1), lambda i, j: (i, j, 0)),
            pl.BlockSpec((1, N, Cin), lambda i, j: (i, 0, 0)),
            pl.BlockSpec((1, qch, 3), lambda i, j: (i, j, 0)),
            pl.BlockSpec((Cin, C), lambda i, j: (0, 0)),
            pl.BlockSpec((1, C), lambda i, j: (0, 0)),
        ],
        out_specs=(
            pl.BlockSpec((1, rows_per_chunk, C), lambda i, j: (i, j, 0)),
            pl.BlockSpec((1, C), lambda i, j: (0, 0)),
            pl.BlockSpec((1, C), lambda i, j: (0, 0)),
        ),
        out_shape=(
            jax.ShapeDtypeStruct((B, RT, C), F32),
            jax.ShapeDtypeStruct((1, C), F32),
            jax.ShapeDtypeStruct((1, C), F32),
        ),
    )(idxf, tab, qT, W, b)


def _bnmm_body(y_ref, sc_ref, sh_ref, w_ref, b_ref, z_ref, s_ref, ss_ref):
    h = jnp.maximum(y_ref[0] * sc_ref[...] + sh_ref[...], 0.0)
    z = jnp.dot(h, w_ref[...], preferred_element_type=F32,
                precision=lax.Precision.DEFAULT) + b_ref[...]
    z_ref[0] = z

    @pl.when((pl.program_id(0) == 0) & (pl.program_id(1) == 0))
    def _():
        s_ref[...] = jnp.zeros_like(s_ref)
        ss_ref[...] = jnp.zeros_like(ss_ref)

    s_ref[...] += jnp.sum(z, 0, keepdims=True)
    ss_ref[...] += jnp.sum(z * z, 0, keepdims=True)


def _bn_mlp(y, scale, shift, W, b, rows_per_chunk):
    B, RT, Cin = y.shape
    C = W.shape[1]
    nch = RT // rows_per_chunk
    return pl.pallas_call(
        _bnmm_body,
        grid=(B, nch),
        in_specs=[
            pl.BlockSpec((1, rows_per_chunk, Cin), lambda i, j: (i, j, 0)),
            pl.BlockSpec((1, Cin), lambda i, j: (0, 0)),
            pl.BlockSpec((1, Cin), lambda i, j: (0, 0)),
            pl.BlockSpec((Cin, C), lambda i, j: (0, 0)),
            pl.BlockSpec((1, C), lambda i, j: (0, 0)),
        ],
        out_specs=(
            pl.BlockSpec((1, rows_per_chunk, C), lambda i, j: (i, j, 0)),
            pl.BlockSpec((1, C), lambda i, j: (0, 0)),
            pl.BlockSpec((1, C), lambda i, j: (0, 0)),
        ),
        out_shape=(
            jax.ShapeDtypeStruct((B, RT, C), F32),
            jax.ShapeDtypeStruct((1, C), F32),
            jax.ShapeDtypeStruct((1, C), F32),
        ),
    )(y, scale, shift, W, b)


def _mm_body(x_ref, w_ref, b_ref, z_ref, s_ref, ss_ref):
    z = jnp.dot(x_ref[0], w_ref[...], preferred_element_type=F32,
                precision=lax.Precision.DEFAULT) + b_ref[...]
    z_ref[0] = z

    @pl.when(pl.program_id(0) == 0)
    def _():
        s_ref[...] = jnp.zeros_like(s_ref)
        ss_ref[...] = jnp.zeros_like(ss_ref)

    s_ref[...] += jnp.sum(z, 0, keepdims=True)
    ss_ref[...] += jnp.sum(z * z, 0, keepdims=True)


def _mm_stats(x, W, b):
    B, RT, Cin = x.shape
    C = W.shape[1]
    return pl.pallas_call(
        _mm_body,
        grid=(B,),
        in_specs=[
            pl.BlockSpec((1, RT, Cin), lambda i: (i, 0, 0)),
            pl.BlockSpec((Cin, C), lambda i: (0, 0)),
            pl.BlockSpec((1, C), lambda i: (0, 0)),
        ],
        out_specs=(
            pl.BlockSpec((1, RT, C), lambda i: (i, 0, 0)),
            pl.BlockSpec((1, C), lambda i: (0, 0)),
            pl.BlockSpec((1, C), lambda i: (0, 0)),
        ),
        out_shape=(
            jax.ShapeDtypeStruct((B, RT, C), F32),
            jax.ShapeDtypeStruct((1, C), F32),
            jax.ShapeDtypeStruct((1, C), F32),
        ),
    )(x, W, b)


def _maxcat_body(y_ref, sc_ref, sh_ref, q_ref, o_ref, *, ksamp, center):
    h = jnp.maximum(y_ref[0] * sc_ref[...] + sh_ref[...], 0.0)
    R, C = h.shape
    nq = R // ksamp
    p = jnp.max(h.reshape(nq, ksamp, C), axis=1)
    q = q_ref[0]
    if center:
        q = q - jnp.mean(q, axis=0, keepdims=True)
    o_ref[0] = jnp.concatenate([q, p], axis=1)


def _max_cat(y, scale, shift, qT, ksamp, q_chunk, center=False):
    B, RT, C = y.shape
    nq_total = RT // ksamp
    nch = nq_total // q_chunk
    rows = q_chunk * ksamp
    return pl.pallas_call(
        functools.partial(_maxcat_body, ksamp=ksamp, center=center),
        grid=(B, nch),
        in_specs=[
            pl.BlockSpec((1, rows, C), lambda i, j: (i, j, 0)),
            pl.BlockSpec((1, C), lambda i, j: (0, 0)),
            pl.BlockSpec((1, C), lambda i, j: (0, 0)),
            pl.BlockSpec((1, q_chunk, 3), lambda i, j: (i, j, 0)),
        ],
        out_specs=pl.BlockSpec((1, q_chunk, C + 3), lambda i, j: (i, j, 0)),
        out_shape=jax.ShapeDtypeStruct((B, nq_total, C + 3), F32),
    )(y, scale, shift, qT)


def _head_body(y_ref, sc_ref, sh_ref,
               w1_ref, b1_ref, g1_ref, be1_ref,
               w2_ref, b2_ref, g2_ref, be2_ref,
               w3_ref, b3_ref, o_ref):
    h = jnp.maximum(y_ref[...] * sc_ref[...] + sh_ref[...], 0.0)
    x = jnp.max(h, axis=1)

    def fc_bn(x, w, b, g, be):
        a = jnp.dot(x, w[...], preferred_element_type=F32,
                    precision=lax.Precision.DEFAULT) + b[...]
        m = jnp.mean(a, axis=0, keepdims=True)
        v = jnp.mean((a - m) ** 2, axis=0, keepdims=True)
        return jnp.maximum(g[...] * (a - m) / jnp.sqrt(v + EPS) + be[...], 0.0)

    x = fc_bn(x, w1_ref, b1_ref, g1_ref, be1_ref)
    x = fc_bn(x, w2_ref, b2_ref, g2_ref, be2_ref)
    o = jnp.dot(x, w3_ref[...], preferred_element_type=F32,
                precision=lax.Precision.DEFAULT) + b3_ref[...]
    o = o - jnp.max(o, axis=1, keepdims=True)
    o_ref[...] = o - jnp.log(jnp.sum(jnp.exp(o), axis=1, keepdims=True))


def _head(y9, scale, shift, fc1, fc2, fc3):
    B = y9.shape[0]
    nc = fc3['W'].shape[1]
    return pl.pallas_call(
        _head_body,
        out_shape=jax.ShapeDtypeStruct((B, nc), F32),
    )(y9, scale, shift,
      fc1['W'], fc1['b'][None, :], fc1['g'][None, :], fc1['be'][None, :],
      fc2['W'], fc2['b'][None, :], fc2['g'][None, :], fc2['be'][None, :],
      fc3['W'], fc3['b'][None, :])


def _finalize(s, ss, n, g, be):
    mean = s / n
    var = ss / n - mean * mean
    scale = g[None, :] / jnp.sqrt(var + EPS)
    shift = be[None, :] - mean * scale
    return scale, shift


def kernel(xyz, params):
    B, _, N = xyz.shape
    xyzT = jnp.transpose(xyz, (0, 2, 1))

    sa1 = params['sa1']
    q1_3, q1_T = _fps(xyz, 512)
    idx1 = _knn(xyz, q1_T, 32, offset=True)
    n1 = B * 512 * 32
    xyzP = jnp.concatenate([xyzT, jnp.zeros((B, N, 13), F32)], -1).reshape(B * N, 16)
    g1 = _sc_gather(xyzP, idx1.reshape(n1)).reshape(B, 512 * 32, 16)
    L = sa1[0]
    W1p = jnp.concatenate([L['W'], jnp.zeros((13, L['W'].shape[1]), F32)], 0)
    y1, s, ss = _g1post(g1, q1_T, W1p, L['b'][None, :], 32, 1024)
    sc, sh = _finalize(s, ss, n1, sa1[0]['g'], sa1[0]['be'])
    L = sa1[1]
    y2, s, ss = _bn_mlp(y1, sc, sh, L['W'], L['b'][None, :], 2048)
    sc, sh = _finalize(s, ss, n1, L['g'], L['be'])
    L = sa1[2]
    y3, s, ss = _bn_mlp(y2, sc, sh, L['W'], L['b'][None, :], 2048)
    sc, sh = _finalize(s, ss, n1, L['g'], L['be'])
    cat1 = _max_cat(y3, sc, sh, q1_T, 32, 64)

    sa2 = params['sa2']
    q2_3, q2_T = _fps(q1_3, 128)
    idx2 = _knn(q1_3, q2_T, 64)
    idx2f = idx2.reshape(B, 128 * 64, 1)
    n2 = B * 128 * 64
    L = sa2[0]
    y4, s, ss = _gather2_mlp(idx2f, cat1, q2_T, L['W'], L['b'][None, :], 64, 2048)
    sc, sh = _finalize(s, ss, n2, L['g'], L['be'])
    L = sa2[1]
    y5, s, ss = _bn_mlp(y4, sc, sh, L['W'], L['b'][None, :], 2048)
    sc, sh = _finalize(s, ss, n2, L['g'], L['be'])
    L = sa2[2]
    y6, s, ss = _bn_mlp(y5, sc, sh, L['W'], L['b'][None, :], 2048)
    sc, sh = _finalize(s, ss, n2, L['g'], L['be'])
    cat2 = _max_cat(y6, sc, sh, q2_T, 64, 128, center=True)

    sa3 = params['sa3']
    n3 = B * 128
    L = sa3[0]
    y7, s, ss = _mm_stats(cat2, L['W'], L['b'][None, :])
    sc, sh = _finalize(s, ss, n3, L['g'], L['be'])
    L = sa3[1]
    y8, s, ss = _bn_mlp(y7, sc, sh, L['W'], L['b'][None, :], 128)
    sc, sh = _finalize(s, ss, n3, L['g'], L['be'])
    L = sa3[2]
    y9, s, ss = _bn_mlp(y8, sc, sh, L['W'], L['b'][None, :], 128)
    sc, sh = _finalize(s, ss, n3, L['g'], L['be'])

    return _head(y9, sc, sh, params['fc1'], params['fc2'], params['fc3'])


# speedup vs baseline: 1.2293x; 1.2293x over previous
import functools

import jax
import jax.numpy as jnp
from jax import lax
from jax.experimental import pallas as pl
from jax.experimental.pallas import tpu as pltpu
from jax.experimental.pallas import tpu_sc as plsc

F32 = jnp.float32
I32 = jnp.int32
HI = lax.Precision.HIGHEST
EPS = 1e-5


def _fps_body(x48_ref, o3_ref, oT_ref, *, npoint, nbatch):
    X = x48_ref[...]
    Bv = nbatch
    Nv = X.shape[1]
    lane = lax.broadcasted_iota(I32, (Bv, Nv), 1)
    lane48 = lax.broadcasted_iota(I32, (3 * Bv, Nv), 1)
    rec_lane = lax.broadcasted_iota(I32, (3 * Bv, npoint), 1)

    def body(i, c):
        dist, far, crec = c
        far48 = jnp.concatenate([far, far, far], 0)
        cents = jnp.sum(jnp.where(lane48 == far48, X, 0.0), 1, keepdims=True)
        crec = jnp.where(rec_lane == i, cents, crec)
        sq = (X - cents) ** 2
        d = (sq[0:Bv, :] + sq[Bv:2 * Bv, :]) + sq[2 * Bv:3 * Bv, :]
        dist = jnp.minimum(dist, d)
        mx = jnp.max(dist, 1, keepdims=True)
        far = jnp.min(jnp.where(dist == mx, lane, Nv), 1, keepdims=True)
        return dist, far, crec

    init = (
        jnp.full((Bv, Nv), 1e10, F32),
        jnp.zeros((Bv, 1), I32),
        jnp.zeros((3 * Bv, npoint), F32),
    )
    _, _, crec = lax.fori_loop(0, npoint, body, init)
    for c in range(3):
        o3_ref[:, c, :] = crec[c * Bv:(c + 1) * Bv, :]
        oT_ref[:, :, c] = crec[c * Bv:(c + 1) * Bv, :]


def _fps(xyz3n, npoint):
    B, _, N = xyz3n.shape
    x48 = jnp.transpose(xyz3n, (1, 0, 2)).reshape(3 * B, N)
    return pl.pallas_call(
        functools.partial(_fps_body, npoint=npoint, nbatch=B),
        out_shape=(
            jax.ShapeDtypeStruct((B, 3, npoint), F32),
            jax.ShapeDtypeStruct((B, npoint, 3), F32),
        ),
    )(x48)


def _sc_gather(table, idx, chunk=2048):
    info = plsc.get_sparse_core_info()
    NW = info.num_cores * info.num_subcores
    R = idx.shape[0]
    D = table.shape[1]
    b_per_w = R // NW
    nch = b_per_w // chunk
    mesh = plsc.VectorSubcoreMesh(core_axis_name="c", subcore_axis_name="s")

    @functools.partial(
        pl.kernel,
        mesh=mesh,
        compiler_params=pltpu.CompilerParams(use_tc_tiling_on_sc=False),
        out_type=jax.ShapeDtypeStruct((R, D), jnp.float32),
        scratch_types=[
            pltpu.VMEM((chunk,), jnp.int32),
            pltpu.VMEM((chunk, D), jnp.float32),
            pltpu.SemaphoreType.DMA,
        ],
    )
    def k(table_hbm, idx_hbm, out_hbm, idx_v, rows_v, sem):
        wid = lax.axis_index("s") * info.num_cores + lax.axis_index("c")
        base = wid * b_per_w
        for j in range(nch):
            off = base + j * chunk
            pltpu.sync_copy(idx_hbm.at[pl.ds(off, chunk)], idx_v)
            pltpu.async_copy(table_hbm.at[idx_v], rows_v, sem).wait()
            pltpu.sync_copy(rows_v, out_hbm.at[pl.ds(off, chunk)])

    return k(table, idx)


def _knn_body(p_ref, q_ref, idx_ref, *, k, offset):
    p = p_ref[0]
    q = q_ref[0]
    NP = q.shape[0]
    N = p.shape[1]
    px = p[0:1, :]
    py = p[1:2, :]
    pz = p[2:3, :]
    qx = q[:, 0:1]
    qy = q[:, 1:2]
    qz = q[:, 2:3]
    sqq = (qx * qx + qy * qy) + qz * qz
    sqp = (px * px + py * py) + pz * pz
    P = jnp.dot(q, p, preferred_element_type=F32,
                precision=lax.Precision.DEFAULT)
    D = (sqq + sqp) - 2.0 * P
    lane = lax.broadcasted_iota(I32, (NP, N), 1)
    kl = lax.broadcasted_iota(I32, (NP, k), 1)

    def body(j, c):
        D, acc = c
        m = jnp.min(D, 1, keepdims=True)
        sel = jnp.min(jnp.where(D == m, lane, N), 1, keepdims=True)
        acc = jnp.where(kl == j, sel, acc)
        D = jnp.where(lane == sel, jnp.inf, D)
        return D, acc

    _, acc = lax.fori_loop(0, k, body, (D, jnp.zeros((NP, k), I32)))
    if offset:
        acc = acc + pl.program_id(0) * N
    idx_ref[0] = acc


def _knn(p3n, qT, k, offset=False):
    B, _, N = p3n.shape
    NP = qT.shape[1]
    return pl.pallas_call(
        functools.partial(_knn_body, k=k, offset=offset),
        grid=(B,),
        in_specs=[
            pl.BlockSpec((1, 3, N), lambda i: (i, 0, 0)),
            pl.BlockSpec((1, NP, 3), lambda i: (i, 0, 0)),
        ],
        out_specs=pl.BlockSpec((1, NP, k), lambda i: (i, 0, 0)),
        out_shape=jax.ShapeDtypeStruct((B, NP, k), I32),
    )(p3n, qT)


def _g1post_body(g_ref, q_ref, w_ref, b_ref, y_ref, s_ref, ss_ref, *, ksamp):
    g = g_ref[0]
    R, Dp = g.shape
    q = q_ref[0]
    nq = q.shape[0]
    qb = jnp.broadcast_to(q[:, None, :], (nq, ksamp, 3)).reshape(R, 3)
    x0 = g - jnp.concatenate([qb, jnp.zeros((R, Dp - 3), F32)], axis=1)
    y = jnp.dot(x0, w_ref[...], preferred_element_type=F32,
                precision=lax.Precision.DEFAULT) + b_ref[...]
    y_ref[0] = y

    @pl.when((pl.program_id(0) == 0) & (pl.program_id(1) == 0))
    def _():
        s_ref[...] = jnp.zeros_like(s_ref)
        ss_ref[...] = jnp.zeros_like(ss_ref)

    s_ref[...] += jnp.sum(y, 0, keepdims=True)
    ss_ref[...] += jnp.sum(y * y, 0, keepdims=True)


def _g1post(g, qT, Wp, b, ksamp, rows_per_chunk):
    B, RT, Dp = g.shape
    C = Wp.shape[1]
    nch = RT // rows_per_chunk
    qch = rows_per_chunk // ksamp
    return pl.pallas_call(
        functools.partial(_g1post_body, ksamp=ksamp),
        grid=(B, nch),
        in_specs=[
            pl.BlockSpec((1, rows_per_chunk, Dp), lambda i, j: (i, j, 0)),
            pl.BlockSpec((1, qch, 3), lambda i, j: (i, j, 0)),
            pl.BlockSpec((Dp, C), lambda i, j: (0, 0)),
            pl.BlockSpec((1, C), lambda i, j: (0, 0)),
        ],
        out_specs=(
            pl.BlockSpec((1, rows_per_chunk, C), lambda i, j: (i, j, 0)),
            pl.BlockSpec((1, C), lambda i, j: (0, 0)),
            pl.BlockSpec((1, C), lambda i, j: (0, 0)),
        ),
        out_shape=(
            jax.ShapeDtypeStruct((B, RT, C), F32),
            jax.ShapeDtypeStruct((1, C), F32),
            jax.ShapeDtypeStruct((1, C), F32),
        ),
    )(g, qT, Wp, b)


def _g2_body(idx_ref, tab_ref, q_ref, w_ref, b_ref, y_ref, s_ref, ss_ref, *, ksamp):
    idxc = idx_ref[0]
    R = idxc.shape[0]
    N = tab_ref.shape[1]
    S = (idxc == lax.broadcasted_iota(I32, (R, N), 1)).astype(F32)
    g = jnp.dot(S, tab_ref[0], preferred_element_type=F32, precision=HI)
    Cin = g.shape[1]
    q = q_ref[0]
    nq = q.shape[0]
    qb = jnp.broadcast_to(q[:, None, :], (nq, ksamp, 3)).reshape(R, 3)
    x = g - jnp.concatenate([qb, jnp.zeros((R, Cin - 3), F32)], axis=1)
    y = jnp.dot(x, w_ref[...], preferred_element_type=F32,
                precision=lax.Precision.DEFAULT) + b_ref[...]
    y_ref[0] = y

    @pl.when((pl.program_id(0) == 0) & (pl.program_id(1) == 0))
    def _():
        s_ref[...] = jnp.zeros_like(s_ref)
        ss_ref[...] = jnp.zeros_like(ss_ref)

    s_ref[...] += jnp.sum(y, 0, keepdims=True)
    ss_ref[...] += jnp.sum(y * y, 0, keepdims=True)


def _gather2_mlp(idxf, tab, qT, W, b, ksamp, rows_per_chunk):
    B, RT, _ = idxf.shape
    N, Cin = tab.shape[1], tab.shape[2]
    C = W.shape[1]
    nch = RT // rows_per_chunk
    qch = rows_per_chunk // ksamp
    return pl.pallas_call(
        functools.partial(_g2_body, ksamp=ksamp),
        grid=(B, nch),
        in_specs=[
            pl.BlockSpec((1, rows_per_chunk, 1), lambda i, j: (i, j, 0)),
            pl.BlockSpec((1, N, Cin), lambda i, j: (i, 0, 0)),
            pl.BlockSpec((1, qch, 3), lambda i, j: (i, j, 0)),
            pl.BlockSpec((Cin, C), lambda i, j: (0, 0)),
            pl.BlockSpec((1, C), lambda i, j: (0, 0)),
        ],
        out_specs=(
            pl.BlockSpec((1, rows_per_chunk, C), lambda i, j: (i, j, 0)),
            pl.BlockSpec((1, C), lambda i, j: (0, 0)),
            pl.BlockSpec((1, C), lambda i, j: (0, 0)),
        ),
        out_shape=(
            jax.ShapeDtypeStruct((B, RT, C), F32),
            jax.ShapeDtypeStruct((1, C), F32),
            jax.ShapeDtypeStruct((1, C), F32),
        ),
    )(idxf, tab, qT, W, b)


def _bnmm_body(y_ref, sc_ref, sh_ref, w_ref, b_ref, z_ref, s_ref, ss_ref):
    h = jnp.maximum(y_ref[0] * sc_ref[...] + sh_ref[...], 0.0)
    z = jnp.dot(h, w_ref[...], preferred_element_type=F32,
                precision=lax.Precision.DEFAULT) + b_ref[...]
    z_ref[0] = z

    @pl.when((pl.program_id(0) == 0) & (pl.program_id(1) == 0))
    def _():
        s_ref[...] = jnp.zeros_like(s_ref)
        ss_ref[...] = jnp.zeros_like(ss_ref)

    s_ref[...] += jnp.sum(z, 0, keepdims=True)
    ss_ref[...] += jnp.sum(z * z, 0, keepdims=True)


def _bn_mlp(y, scale, shift, W, b, rows_per_chunk):
    B, RT, Cin = y.shape
    C = W.shape[1]
    nch = RT // rows_per_chunk
    return pl.pallas_call(
        _bnmm_body,
        grid=(B, nch),
        in_specs=[
            pl.BlockSpec((1, rows_per_chunk, Cin), lambda i, j: (i, j, 0)),
            pl.BlockSpec((1, Cin), lambda i, j: (0, 0)),
            pl.BlockSpec((1, Cin), lambda i, j: (0, 0)),
            pl.BlockSpec((Cin, C), lambda i, j: (0, 0)),
            pl.BlockSpec((1, C), lambda i, j: (0, 0)),
        ],
        out_specs=(
            pl.BlockSpec((1, rows_per_chunk, C), lambda i, j: (i, j, 0)),
            pl.BlockSpec((1, C), lambda i, j: (0, 0)),
            pl.BlockSpec((1, C), lambda i, j: (0, 0)),
        ),
        out_shape=(
            jax.ShapeDtypeStruct((B, RT, C), F32),
            jax.ShapeDtypeStruct((1, C), F32),
            jax.ShapeDtypeStruct((1, C), F32),
        ),
    )(y, scale, shift, W, b)


def _mm_body(x_ref, w_ref, b_ref, z_ref, s_ref, ss_ref):
    z = jnp.dot(x_ref[0], w_ref[...], preferred_element_type=F32,
                precision=lax.Precision.DEFAULT) + b_ref[...]
    z_ref[0] = z

    @pl.when(pl.program_id(0) == 0)
    def _():
        s_ref[...] = jnp.zeros_like(s_ref)
        ss_ref[...] = jnp.zeros_like(ss_ref)

    s_ref[...] += jnp.sum(z, 0, keepdims=True)
    ss_ref[...] += jnp.sum(z * z, 0, keepdims=True)


def _mm_stats(x, W, b):
    B, RT, Cin = x.shape
    C = W.shape[1]
    return pl.pallas_call(
        _mm_body,
        grid=(B,),
        in_specs=[
            pl.BlockSpec((1, RT, Cin), lambda i: (i, 0, 0)),
            pl.BlockSpec((Cin, C), lambda i: (0, 0)),
            pl.BlockSpec((1, C), lambda i: (0, 0)),
        ],
        out_specs=(
            pl.BlockSpec((1, RT, C), lambda i: (i, 0, 0)),
            pl.BlockSpec((1, C), lambda i: (0, 0)),
            pl.BlockSpec((1, C), lambda i: (0, 0)),
        ),
        out_shape=(
            jax.ShapeDtypeStruct((B, RT, C), F32),
            jax.ShapeDtypeStruct((1, C), F32),
            jax.ShapeDtypeStruct((1, C), F32),
        ),
    )(x, W, b)


def _maxcat_body(y_ref, sc_ref, sh_ref, q_ref, o_ref, *, ksamp, center):
    h = jnp.maximum(y_ref[0] * sc_ref[...] + sh_ref[...], 0.0)
    R, C = h.shape
    nq = R // ksamp
    p = jnp.max(h.reshape(nq, ksamp, C), axis=1)
    q = q_ref[0]
    if center:
        q = q - jnp.mean(q, axis=0, keepdims=True)
    o_ref[0] = jnp.concatenate([q, p], axis=1)


def _max_cat(y, scale, shift, qT, ksamp, q_chunk, center=False):
    B, RT, C = y.shape
    nq_total = RT // ksamp
    nch = nq_total // q_chunk
    rows = q_chunk * ksamp
    return pl.pallas_call(
        functools.partial(_maxcat_body, ksamp=ksamp, center=center),
        grid=(B, nch),
        in_specs=[
            pl.BlockSpec((1, rows, C), lambda i, j: (i, j, 0)),
            pl.BlockSpec((1, C), lambda i, j: (0, 0)),
            pl.BlockSpec((1, C), lambda i, j: (0, 0)),
            pl.BlockSpec((1, q_chunk, 3), lambda i, j: (i, j, 0)),
        ],
        out_specs=pl.BlockSpec((1, q_chunk, C + 3), lambda i, j: (i, j, 0)),
        out_shape=jax.ShapeDtypeStruct((B, nq_total, C + 3), F32),
    )(y, scale, shift, qT)


def _head_body(y_ref, sc_ref, sh_ref,
               w1_ref, b1_ref, g1_ref, be1_ref,
               w2_ref, b2_ref, g2_ref, be2_ref,
               w3_ref, b3_ref, o_ref):
    h = jnp.maximum(y_ref[...] * sc_ref[...] + sh_ref[...], 0.0)
    x = jnp.max(h, axis=1)

    def fc_bn(x, w, b, g, be):
        a = jnp.dot(x, w[...], preferred_element_type=F32,
                    precision=lax.Precision.DEFAULT) + b[...]
        m = jnp.mean(a, axis=0, keepdims=True)
        v = jnp.mean((a - m) ** 2, axis=0, keepdims=True)
        return jnp.maximum(g[...] * (a - m) / jnp.sqrt(v + EPS) + be[...], 0.0)

    x = fc_bn(x, w1_ref, b1_ref, g1_ref, be1_ref)
    x = fc_bn(x, w2_ref, b2_ref, g2_ref, be2_ref)
    o = jnp.dot(x, w3_ref[...], preferred_element_type=F32,
                precision=lax.Precision.DEFAULT) + b3_ref[...]
    o = o - jnp.max(o, axis=1, keepdims=True)
    o_ref[...] = o - jnp.log(jnp.sum(jnp.exp(o), axis=1, keepdims=True))


def _head(y9, scale, shift, fc1, fc2, fc3):
    B = y9.shape[0]
    nc = fc3['W'].shape[1]
    return pl.pallas_call(
        _head_body,
        out_shape=jax.ShapeDtypeStruct((B, nc), F32),
    )(y9, scale, shift,
      fc1['W'], fc1['b'][None, :], fc1['g'][None, :], fc1['be'][None, :],
      fc2['W'], fc2['b'][None, :], fc2['g'][None, :], fc2['be'][None, :],
      fc3['W'], fc3['b'][None, :])


def _finalize(s, ss, n, g, be):
    mean = s / n
    var = ss / n - mean * mean
    scale = g[None, :] / jnp.sqrt(var + EPS)
    shift = be[None, :] - mean * scale
    return scale, shift


def kernel(xyz, params):
    B, _, N = xyz.shape
    xyzT = jnp.transpose(xyz, (0, 2, 1))

    sa1 = params['sa1']
    q1_3, q1_T = _fps(xyz, 512)
    idx1 = _knn(xyz, q1_T, 32, offset=True)
    n1 = B * 512 * 32
    xyzP = jnp.concatenate([xyzT, jnp.zeros((B, N, 13), F32)], -1).reshape(B * N, 16)
    g1 = _sc_gather(xyzP, idx1.reshape(n1)).reshape(B, 512 * 32, 16)
    L = sa1[0]
    W1p = jnp.concatenate([L['W'], jnp.zeros((13, L['W'].shape[1]), F32)], 0)
    y1, s, ss = _g1post(g1, q1_T, W1p, L['b'][None, :], 32, 1024)
    sc, sh = _finalize(s, ss, n1, sa1[0]['g'], sa1[0]['be'])
    L = sa1[1]
    y2, s, ss = _bn_mlp(y1, sc, sh, L['W'], L['b'][None, :], 2048)
    sc, sh = _finalize(s, ss, n1, L['g'], L['be'])
    L = sa1[2]
    y3, s, ss = _bn_mlp(y2, sc, sh, L['W'], L['b'][None, :], 2048)
    sc, sh = _finalize(s, ss, n1, L['g'], L['be'])
    cat1 = _max_cat(y3, sc, sh, q1_T, 32, 64)

    sa2 = params['sa2']
    q2_3, q2_T = _fps(q1_3, 128)
    idx2 = _knn(q1_3, q2_T, 64)
    idx2f = idx2.reshape(B, 128 * 64, 1)
    n2 = B * 128 * 64
    L = sa2[0]
    y4, s, ss = _gather2_mlp(idx2f, cat1, q2_T, L['W'], L['b'][None, :], 64, 2048)
    sc, sh = _finalize(s, ss, n2, L['g'], L['be'])
    L = sa2[1]
    y5, s, ss = _bn_mlp(y4, sc, sh, L['W'], L['b'][None, :], 2048)
    sc, sh = _finalize(s, ss, n2, L['g'], L['be'])
    L = sa2[2]
    y6, s, ss = _bn_mlp(y5, sc, sh, L['W'], L['b'][None, :], 2048)
    sc, sh = _finalize(s, ss, n2, L['g'], L['be'])
    cat2 = _max_cat(y6, sc, sh, q2_T, 64, 128, center=True)

    sa3 = params['sa3']
    n3 = B * 128
    L = sa3[0]
    y7, s, ss = _mm_stats(cat2, L['W'], L['b'][None, :])
    sc, sh = _finalize(s, ss, n3, L['g'], L['be'])
    L = sa3[1]
    y8, s, ss = _bn_mlp(y7, sc, sh, L['W'], L['b'][None, :], 128)
    sc, sh = _finalize(s, ss, n3, L['g'], L['be'])
    L = sa3[2]
    y9, s, ss = _bn_mlp(y8, sc, sh, L['W'], L['b'][None, :], 128)
    sc, sh = _finalize(s, ss, n3, L['g'], L['be'])

    return _head(y9, sc, sh, params['fc1'], params['fc2'], params['fc3'])


# speedup vs baseline: 1.2856x; 1.0458x over previous
import functools

import jax
import jax.numpy as jnp
from jax import lax
from jax.experimental import pallas as pl
from jax.experimental.pallas import tpu as pltpu
from jax.experimental.pallas import tpu_sc as plsc

F32 = jnp.float32
I32 = jnp.int32
HI = lax.Precision.HIGHEST
EPS = 1e-5


def _fps_body(x48_ref, o3_ref, oT_ref, *, npoint, nbatch):
    X = x48_ref[...]
    Bv = nbatch
    Nv = X.shape[1]
    lane = lax.broadcasted_iota(I32, (Bv, Nv), 1)
    lane48 = lax.broadcasted_iota(I32, (3 * Bv, Nv), 1)
    rec_lane = lax.broadcasted_iota(I32, (3 * Bv, npoint), 1)

    def body(i, c):
        dist, far, crec = c
        far48 = jnp.concatenate([far, far, far], 0)
        cents = jnp.sum(jnp.where(lane48 == far48, X, 0.0), 1, keepdims=True)
        crec = jnp.where(rec_lane == i, cents, crec)
        sq = (X - cents) ** 2
        d = (sq[0:Bv, :] + sq[Bv:2 * Bv, :]) + sq[2 * Bv:3 * Bv, :]
        dist = jnp.minimum(dist, d)
        mx = jnp.max(dist, 1, keepdims=True)
        far = jnp.min(jnp.where(dist == mx, lane, Nv), 1, keepdims=True)
        return dist, far, crec

    init = (
        jnp.full((Bv, Nv), 1e10, F32),
        jnp.zeros((Bv, 1), I32),
        jnp.zeros((3 * Bv, npoint), F32),
    )
    _, _, crec = lax.fori_loop(0, npoint, body, init)
    for c in range(3):
        o3_ref[:, c, :] = crec[c * Bv:(c + 1) * Bv, :]
        oT_ref[:, :, c] = crec[c * Bv:(c + 1) * Bv, :]


def _fps(xyz3n, npoint):
    B, _, N = xyz3n.shape
    x48 = jnp.transpose(xyz3n, (1, 0, 2)).reshape(3 * B, N)
    return pl.pallas_call(
        functools.partial(_fps_body, npoint=npoint, nbatch=B),
        out_shape=(
            jax.ShapeDtypeStruct((B, 3, npoint), F32),
            jax.ShapeDtypeStruct((B, npoint, 3), F32),
        ),
    )(x48)


def _sc_gather(table, idx, chunk=2048):
    info = plsc.get_sparse_core_info()
    NW = info.num_cores * info.num_subcores
    R = idx.shape[0]
    D = table.shape[1]
    b_per_w = R // NW
    nch = b_per_w // chunk
    mesh = plsc.VectorSubcoreMesh(core_axis_name="c", subcore_axis_name="s")

    @functools.partial(
        pl.kernel,
        mesh=mesh,
        compiler_params=pltpu.CompilerParams(use_tc_tiling_on_sc=False),
        out_type=jax.ShapeDtypeStruct((R, D), jnp.float32),
        scratch_types=[
            pltpu.VMEM((chunk,), jnp.int32),
            pltpu.VMEM((chunk, D), jnp.float32),
            pltpu.SemaphoreType.DMA,
        ],
    )
    def k(table_hbm, idx_hbm, out_hbm, idx_v, rows_v, sem):
        wid = lax.axis_index("s") * info.num_cores + lax.axis_index("c")
        base = wid * b_per_w
        for j in range(nch):
            off = base + j * chunk
            pltpu.sync_copy(idx_hbm.at[pl.ds(off, chunk)], idx_v)
            pltpu.async_copy(table_hbm.at[idx_v], rows_v, sem).wait()
            pltpu.sync_copy(rows_v, out_hbm.at[pl.ds(off, chunk)])

    return k(table, idx)


def _knn_body(p_ref, q_ref, idx_ref, *, k, offset):
    p = p_ref[0]
    q = q_ref[0]
    NP = q.shape[0]
    N = p.shape[1]
    px = p[0:1, :]
    py = p[1:2, :]
    pz = p[2:3, :]
    qx = q[:, 0:1]
    qy = q[:, 1:2]
    qz = q[:, 2:3]
    sqq = (qx * qx + qy * qy) + qz * qz
    sqp = (px * px + py * py) + pz * pz
    P = jnp.dot(q, p, preferred_element_type=F32,
                precision=lax.Precision.DEFAULT)
    D = (sqq + sqp) - 2.0 * P
    lane = lax.broadcasted_iota(I32, (NP, N), 1)
    kl = lax.broadcasted_iota(I32, (NP, k), 1)

    def body(j, c):
        D, acc = c
        m = jnp.min(D, 1, keepdims=True)
        sel = jnp.min(jnp.where(D == m, lane, N), 1, keepdims=True)
        acc = jnp.where(kl == j, sel, acc)
        D = jnp.where(lane == sel, jnp.inf, D)
        return D, acc

    _, acc = lax.fori_loop(0, k, body, (D, jnp.zeros((NP, k), I32)))
    if offset:
        acc = acc + pl.program_id(0) * N
    idx_ref[0] = acc


def _knn_all_body(p_ref, q_ref, idx_ref, *, k, nbatch):
    NP = q_ref.shape[1]
    N = p_ref.shape[2]
    rows = nbatch * NP
    Ps, sqqs, sqps = [], [], []
    for b in range(nbatch):
        qb = q_ref[b]
        pb = p_ref[b]
        Ps.append(jnp.dot(qb, pb, preferred_element_type=F32,
                          precision=lax.Precision.DEFAULT))
        qx, qy, qz = qb[:, 0:1], qb[:, 1:2], qb[:, 2:3]
        px, py, pz = pb[0:1, :], pb[1:2, :], pb[2:3, :]
        sqqs.append((qx * qx + qy * qy) + qz * qz)
        sqps.append(jnp.broadcast_to((px * px + py * py) + pz * pz, (NP, N)))
    P = jnp.concatenate(Ps, 0)
    sqq = jnp.concatenate(sqqs, 0)
    sqp = jnp.concatenate(sqps, 0)
    D = (sqq + sqp) - 2.0 * P
    lane = lax.broadcasted_iota(I32, (rows, N), 1)
    kl = lax.broadcasted_iota(I32, (rows, k), 1)

    def body(j, c):
        D, acc = c
        m = jnp.min(D, 1, keepdims=True)
        sel = jnp.min(jnp.where(D == m, lane, N), 1, keepdims=True)
        acc = jnp.where(kl == j, sel, acc)
        D = jnp.where(lane == sel, jnp.inf, D)
        return D, acc

    _, acc = lax.fori_loop(0, k, body, (D, jnp.zeros((rows, k), I32)))
    idx_ref[...] = acc.reshape(nbatch, NP, k)


def _knn_all(p3n, qT, k):
    B, _, N = p3n.shape
    NP = qT.shape[1]
    return pl.pallas_call(
        functools.partial(_knn_all_body, k=k, nbatch=B),
        out_shape=jax.ShapeDtypeStruct((B, NP, k), I32),
    )(p3n, qT)


def _knn(p3n, qT, k, offset=False):
    B, _, N = p3n.shape
    NP = qT.shape[1]
    return pl.pallas_call(
        functools.partial(_knn_body, k=k, offset=offset),
        grid=(B,),
        in_specs=[
            pl.BlockSpec((1, 3, N), lambda i: (i, 0, 0)),
            pl.BlockSpec((1, NP, 3), lambda i: (i, 0, 0)),
        ],
        out_specs=pl.BlockSpec((1, NP, k), lambda i: (i, 0, 0)),
        out_shape=jax.ShapeDtypeStruct((B, NP, k), I32),
    )(p3n, qT)


def _g1post_body(g_ref, q_ref, w_ref, b_ref, y_ref, s_ref, ss_ref, *, ksamp):
    g = g_ref[0]
    R, Dp = g.shape
    q = q_ref[0]
    nq = q.shape[0]
    qb = jnp.broadcast_to(q[:, None, :], (nq, ksamp, 3)).reshape(R, 3)
    x0 = g - jnp.concatenate([qb, jnp.zeros((R, Dp - 3), F32)], axis=1)
    y = jnp.dot(x0, w_ref[...], preferred_element_type=F32,
                precision=lax.Precision.DEFAULT) + b_ref[...]
    y_ref[0] = y

    @pl.when((pl.program_id(0) == 0) & (pl.program_id(1) == 0))
    def _():
        s_ref[...] = jnp.zeros_like(s_ref)
        ss_ref[...] = jnp.zeros_like(ss_ref)

    s_ref[...] += jnp.sum(y, 0, keepdims=True)
    ss_ref[...] += jnp.sum(y * y, 0, keepdims=True)


def _g1post(g, qT, Wp, b, ksamp, rows_per_chunk):
    B, RT, Dp = g.shape
    C = Wp.shape[1]
    nch = RT // rows_per_chunk
    qch = rows_per_chunk // ksamp
    return pl.pallas_call(
        functools.partial(_g1post_body, ksamp=ksamp),
        grid=(B, nch),
        in_specs=[
            pl.BlockSpec((1, rows_per_chunk, Dp), lambda i, j: (i, j, 0)),
            pl.BlockSpec((1, qch, 3), lambda i, j: (i, j, 0)),
            pl.BlockSpec((Dp, C), lambda i, j: (0, 0)),
            pl.BlockSpec((1, C), lambda i, j: (0, 0)),
        ],
        out_specs=(
            pl.BlockSpec((1, rows_per_chunk, C), lambda i, j: (i, j, 0)),
            pl.BlockSpec((1, C), lambda i, j: (0, 0)),
            pl.BlockSpec((1, C), lambda i, j: (0, 0)),
        ),
        out_shape=(
            jax.ShapeDtypeStruct((B, RT, C), F32),
            jax.ShapeDtypeStruct((1, C), F32),
            jax.ShapeDtypeStruct((1, C), F32),
        ),
    )(g, qT, Wp, b)


def _g2_body(idx_ref, tab_ref, q_ref, w_ref, b_ref, y_ref, s_ref, ss_ref, *, ksamp):
    idxc = idx_ref[0]
    R = idxc.shape[0]
    N = tab_ref.shape[1]
    S = (idxc == lax.broadcasted_iota(I32, (R, N), 1)).astype(F32)
    g = jnp.dot(S, tab_ref[0], preferred_element_type=F32, precision=HI)
    Cin = g.shape[1]
    q = q_ref[0]
    nq = q.shape[0]
    qb = jnp.broadcast_to(q[:, None, :], (nq, ksamp, 3)).reshape(R, 3)
    x = g - jnp.concatenate([qb, jnp.zeros((R, Cin - 3), F32)], axis=1)
    y = jnp.dot(x, w_ref[...], preferred_element_type=F32,
                precision=lax.Precision.DEFAULT) + b_ref[...]
    y_ref[0] = y

    @pl.when((pl.program_id(0) == 0) & (pl.program_id(1) == 0))
    def _():
        s_ref[...] = jnp.zeros_like(s_ref)
        ss_ref[...] = jnp.zeros_like(ss_ref)

    s_ref[...] += jnp.sum(y, 0, keepdims=True)
    ss_ref[...] += jnp.sum(y * y, 0, keepdims=True)


def _gather2_mlp(idxf, tab, qT, W, b, ksamp, rows_per_chunk):
    B, RT, _ = idxf.shape
    N, Cin = tab.shape[1], tab.shape[2]
    C = W.shape[1]
    nch = RT // rows_per_chunk
    qch = rows_per_chunk // ksamp
    return pl.pallas_call(
        functools.partial(_g2_body, ksamp=ksamp),
        grid=(B, nch),
        in_specs=[
            pl.BlockSpec((1, rows_per_chunk, 1), lambda i, j: (i, j, 0)),
            pl.BlockSpec((1, N, Cin), lambda i, j: (i, 0, 0)),
            pl.BlockSpec((1, qch, 3), lambda i, j: (i, j, 0)),
            pl.BlockSpec((Cin, C), lambda i, j: (0, 0)),
            pl.BlockSpec((1, C), lambda i, j: (0, 0)),
        ],
        out_specs=(
            pl.BlockSpec((1, rows_per_chunk, C), lambda i, j: (i, j, 0)),
            pl.BlockSpec((1, C), lambda i, j: (0, 0)),
            pl.BlockSpec((1, C), lambda i, j: (0, 0)),
        ),
        out_shape=(
            jax.ShapeDtypeStruct((B, RT, C), F32),
            jax.ShapeDtypeStruct((1, C), F32),
            jax.ShapeDtypeStruct((1, C), F32),
        ),
    )(idxf, tab, qT, W, b)


def _bnmm_body(y_ref, sc_ref, sh_ref, w_ref, b_ref, z_ref, s_ref, ss_ref):
    h = jnp.maximum(y_ref[0] * sc_ref[...] + sh_ref[...], 0.0)
    z = jnp.dot(h, w_ref[...], preferred_element_type=F32,
                precision=lax.Precision.DEFAULT) + b_ref[...]
    z_ref[0] = z

    @pl.when((pl.program_id(0) == 0) & (pl.program_id(1) == 0))
    def _():
        s_ref[...] = jnp.zeros_like(s_ref)
        ss_ref[...] = jnp.zeros_like(ss_ref)

    s_ref[...] += jnp.sum(z, 0, keepdims=True)
    ss_ref[...] += jnp.sum(z * z, 0, keepdims=True)


def _bn_mlp(y, scale, shift, W, b, rows_per_chunk):
    B, RT, Cin = y.shape
    C = W.shape[1]
    nch = RT // rows_per_chunk
    return pl.pallas_call(
        _bnmm_body,
        grid=(B, nch),
        in_specs=[
            pl.BlockSpec((1, rows_per_chunk, Cin), lambda i, j: (i, j, 0)),
            pl.BlockSpec((1, Cin), lambda i, j: (0, 0)),
            pl.BlockSpec((1, Cin), lambda i, j: (0, 0)),
            pl.BlockSpec((Cin, C), lambda i, j: (0, 0)),
            pl.BlockSpec((1, C), lambda i, j: (0, 0)),
        ],
        out_specs=(
            pl.BlockSpec((1, rows_per_chunk, C), lambda i, j: (i, j, 0)),
            pl.BlockSpec((1, C), lambda i, j: (0, 0)),
            pl.BlockSpec((1, C), lambda i, j: (0, 0)),
        ),
        out_shape=(
            jax.ShapeDtypeStruct((B, RT, C), F32),
            jax.ShapeDtypeStruct((1, C), F32),
            jax.ShapeDtypeStruct((1, C), F32),
        ),
    )(y, scale, shift, W, b)


def _mm_body(x_ref, w_ref, b_ref, z_ref, s_ref, ss_ref):
    z = jnp.dot(x_ref[0], w_ref[...], preferred_element_type=F32,
                precision=lax.Precision.DEFAULT) + b_ref[...]
    z_ref[0] = z

    @pl.when(pl.program_id(0) == 0)
    def _():
        s_ref[...] = jnp.zeros_like(s_ref)
        ss_ref[...] = jnp.zeros_like(ss_ref)

    s_ref[...] += jnp.sum(z, 0, keepdims=True)
    ss_ref[...] += jnp.sum(z * z, 0, keepdims=True)


def _mm_stats(x, W, b):
    B, RT, Cin = x.shape
    C = W.shape[1]
    return pl.pallas_call(
        _mm_body,
        grid=(B,),
        in_specs=[
            pl.BlockSpec((1, RT, Cin), lambda i: (i, 0, 0)),
            pl.BlockSpec((Cin, C), lambda i: (0, 0)),
            pl.BlockSpec((1, C), lambda i: (0, 0)),
        ],
        out_specs=(
            pl.BlockSpec((1, RT, C), lambda i: (i, 0, 0)),
            pl.BlockSpec((1, C), lambda i: (0, 0)),
            pl.BlockSpec((1, C), lambda i: (0, 0)),
        ),
        out_shape=(
            jax.ShapeDtypeStruct((B, RT, C), F32),
            jax.ShapeDtypeStruct((1, C), F32),
            jax.ShapeDtypeStruct((1, C), F32),
        ),
    )(x, W, b)


def _maxcat_body(y_ref, sc_ref, sh_ref, q_ref, o_ref, *, ksamp, center):
    h = jnp.maximum(y_ref[0] * sc_ref[...] + sh_ref[...], 0.0)
    R, C = h.shape
    nq = R // ksamp
    p = jnp.max(h.reshape(nq, ksamp, C), axis=1)
    q = q_ref[0]
    if center:
        q = q - jnp.mean(q, axis=0, keepdims=True)
    o_ref[0] = jnp.concatenate([q, p], axis=1)


def _max_cat(y, scale, shift, qT, ksamp, q_chunk, center=False):
    B, RT, C = y.shape
    nq_total = RT // ksamp
    nch = nq_total // q_chunk
    rows = q_chunk * ksamp
    return pl.pallas_call(
        functools.partial(_maxcat_body, ksamp=ksamp, center=center),
        grid=(B, nch),
        in_specs=[
            pl.BlockSpec((1, rows, C), lambda i, j: (i, j, 0)),
            pl.BlockSpec((1, C), lambda i, j: (0, 0)),
            pl.BlockSpec((1, C), lambda i, j: (0, 0)),
            pl.BlockSpec((1, q_chunk, 3), lambda i, j: (i, j, 0)),
        ],
        out_specs=pl.BlockSpec((1, q_chunk, C + 3), lambda i, j: (i, j, 0)),
        out_shape=jax.ShapeDtypeStruct((B, nq_total, C + 3), F32),
    )(y, scale, shift, qT)


def _head_body(y_ref, sc_ref, sh_ref,
               w1_ref, b1_ref, g1_ref, be1_ref,
               w2_ref, b2_ref, g2_ref, be2_ref,
               w3_ref, b3_ref, o_ref):
    h = jnp.maximum(y_ref[...] * sc_ref[...] + sh_ref[...], 0.0)
    x = jnp.max(h, axis=1)

    def fc_bn(x, w, b, g, be):
        a = jnp.dot(x, w[...], preferred_element_type=F32,
                    precision=lax.Precision.DEFAULT) + b[...]
        m = jnp.mean(a, axis=0, keepdims=True)
        v = jnp.mean((a - m) ** 2, axis=0, keepdims=True)
        return jnp.maximum(g[...] * (a - m) / jnp.sqrt(v + EPS) + be[...], 0.0)

    x = fc_bn(x, w1_ref, b1_ref, g1_ref, be1_ref)
    x = fc_bn(x, w2_ref, b2_ref, g2_ref, be2_ref)
    o = jnp.dot(x, w3_ref[...], preferred_element_type=F32,
                precision=lax.Precision.DEFAULT) + b3_ref[...]
    o = o - jnp.max(o, axis=1, keepdims=True)
    o_ref[...] = o - jnp.log(jnp.sum(jnp.exp(o), axis=1, keepdims=True))


def _head(y9, scale, shift, fc1, fc2, fc3):
    B = y9.shape[0]
    nc = fc3['W'].shape[1]
    return pl.pallas_call(
        _head_body,
        out_shape=jax.ShapeDtypeStruct((B, nc), F32),
    )(y9, scale, shift,
      fc1['W'], fc1['b'][None, :], fc1['g'][None, :], fc1['be'][None, :],
      fc2['W'], fc2['b'][None, :], fc2['g'][None, :], fc2['be'][None, :],
      fc3['W'], fc3['b'][None, :])


def _finalize(s, ss, n, g, be):
    mean = s / n
    var = ss / n - mean * mean
    scale = g[None, :] / jnp.sqrt(var + EPS)
    shift = be[None, :] - mean * scale
    return scale, shift


def kernel(xyz, params):
    B, _, N = xyz.shape
    xyzT = jnp.transpose(xyz, (0, 2, 1))

    sa1 = params['sa1']
    q1_3, q1_T = _fps(xyz, 512)
    idx1 = _knn(xyz, q1_T, 32, offset=True)
    n1 = B * 512 * 32
    xyzP = jnp.concatenate([xyzT, jnp.zeros((B, N, 13), F32)], -1).reshape(B * N, 16)
    g1 = _sc_gather(xyzP, idx1.reshape(n1)).reshape(B, 512 * 32, 16)
    L = sa1[0]
    W1p = jnp.concatenate([L['W'], jnp.zeros((13, L['W'].shape[1]), F32)], 0)
    y1, s, ss = _g1post(g1, q1_T, W1p, L['b'][None, :], 32, 1024)
    sc, sh = _finalize(s, ss, n1, sa1[0]['g'], sa1[0]['be'])
    L = sa1[1]
    y2, s, ss = _bn_mlp(y1, sc, sh, L['W'], L['b'][None, :], 2048)
    sc, sh = _finalize(s, ss, n1, L['g'], L['be'])
    L = sa1[2]
    y3, s, ss = _bn_mlp(y2, sc, sh, L['W'], L['b'][None, :], 2048)
    sc, sh = _finalize(s, ss, n1, L['g'], L['be'])
    cat1 = _max_cat(y3, sc, sh, q1_T, 32, 64)

    sa2 = params['sa2']
    q2_3, q2_T = _fps(q1_3, 128)
    idx2 = _knn_all(q1_3, q2_T, 64)
    idx2f = idx2.reshape(B, 128 * 64, 1)
    n2 = B * 128 * 64
    L = sa2[0]
    y4, s, ss = _gather2_mlp(idx2f, cat1, q2_T, L['W'], L['b'][None, :], 64, 2048)
    sc, sh = _finalize(s, ss, n2, L['g'], L['be'])
    L = sa2[1]
    y5, s, ss = _bn_mlp(y4, sc, sh, L['W'], L['b'][None, :], 2048)
    sc, sh = _finalize(s, ss, n2, L['g'], L['be'])
    L = sa2[2]
    y6, s, ss = _bn_mlp(y5, sc, sh, L['W'], L['b'][None, :], 2048)
    sc, sh = _finalize(s, ss, n2, L['g'], L['be'])
    cat2 = _max_cat(y6, sc, sh, q2_T, 64, 128, center=True)

    sa3 = params['sa3']
    n3 = B * 128
    L = sa3[0]
    y7, s, ss = _mm_stats(cat2, L['W'], L['b'][None, :])
    sc, sh = _finalize(s, ss, n3, L['g'], L['be'])
    L = sa3[1]
    y8, s, ss = _bn_mlp(y7, sc, sh, L['W'], L['b'][None, :], 128)
    sc, sh = _finalize(s, ss, n3, L['g'], L['be'])
    L = sa3[2]
    y9, s, ss = _bn_mlp(y8, sc, sh, L['W'], L['b'][None, :], 128)
    sc, sh = _finalize(s, ss, n3, L['g'], L['be'])

    return _head(y9, sc, sh, params['fc1'], params['fc2'], params['fc3'])


# speedup vs baseline: 1.4191x; 1.1038x over previous
import functools

import jax
import jax.numpy as jnp
from jax import lax
from jax.experimental import pallas as pl
from jax.experimental.pallas import tpu as pltpu
from jax.experimental.pallas import tpu_sc as plsc

F32 = jnp.float32
I32 = jnp.int32
HI = lax.Precision.HIGHEST
EPS = 1e-5


def _fps_body(x48_ref, o3_ref, oT_ref, *, npoint, nbatch):
    X = x48_ref[...]
    Bv = nbatch
    Nv = X.shape[1]
    lane = lax.broadcasted_iota(I32, (Bv, Nv), 1)
    lane48 = lax.broadcasted_iota(I32, (3 * Bv, Nv), 1)
    rec_lane = lax.broadcasted_iota(I32, (3 * Bv, npoint), 1)

    def body(i, c):
        dist, far, crec = c
        far48 = jnp.concatenate([far, far, far], 0)
        cents = jnp.sum(jnp.where(lane48 == far48, X, 0.0), 1, keepdims=True)
        crec = jnp.where(rec_lane == i, cents, crec)
        sq = (X - cents) ** 2
        d = (sq[0:Bv, :] + sq[Bv:2 * Bv, :]) + sq[2 * Bv:3 * Bv, :]
        dist = jnp.minimum(dist, d)
        mx = jnp.max(dist, 1, keepdims=True)
        far = jnp.min(jnp.where(dist == mx, lane, Nv), 1, keepdims=True)
        return dist, far, crec

    init = (
        jnp.full((Bv, Nv), 1e10, F32),
        jnp.zeros((Bv, 1), I32),
        jnp.zeros((3 * Bv, npoint), F32),
    )
    _, _, crec = lax.fori_loop(0, npoint, body, init)
    for c in range(3):
        o3_ref[:, c, :] = crec[c * Bv:(c + 1) * Bv, :]
        oT_ref[:, :, c] = crec[c * Bv:(c + 1) * Bv, :]


def _fps(xyz3n, npoint):
    B, _, N = xyz3n.shape
    x48 = jnp.transpose(xyz3n, (1, 0, 2)).reshape(3 * B, N)
    return pl.pallas_call(
        functools.partial(_fps_body, npoint=npoint, nbatch=B),
        out_shape=(
            jax.ShapeDtypeStruct((B, 3, npoint), F32),
            jax.ShapeDtypeStruct((B, npoint, 3), F32),
        ),
    )(x48)


def _sc_gather(table, idx, chunk=2048):
    info = plsc.get_sparse_core_info()
    NW = info.num_cores * info.num_subcores
    R = idx.shape[0]
    D = table.shape[1]
    b_per_w = R // NW
    nch = b_per_w // chunk
    mesh = plsc.VectorSubcoreMesh(core_axis_name="c", subcore_axis_name="s")

    @functools.partial(
        pl.kernel,
        mesh=mesh,
        compiler_params=pltpu.CompilerParams(use_tc_tiling_on_sc=False),
        out_type=jax.ShapeDtypeStruct((R, D), jnp.float32),
        scratch_types=[
            pltpu.VMEM((chunk,), jnp.int32),
            pltpu.VMEM((chunk, D), jnp.float32),
            pltpu.SemaphoreType.DMA,
        ],
    )
    def k(table_hbm, idx_hbm, out_hbm, idx_v, rows_v, sem):
        wid = lax.axis_index("s") * info.num_cores + lax.axis_index("c")
        base = wid * b_per_w
        for j in range(nch):
            off = base + j * chunk
            pltpu.sync_copy(idx_hbm.at[pl.ds(off, chunk)], idx_v)
            pltpu.async_copy(table_hbm.at[idx_v], rows_v, sem).wait()
            pltpu.sync_copy(rows_v, out_hbm.at[pl.ds(off, chunk)])

    return k(table, idx)


def _knn_body(p_ref, q_ref, idx_ref, *, k, offset):
    p = p_ref[0]
    q = q_ref[0]
    NP = q.shape[0]
    N = p.shape[1]
    px = p[0:1, :]
    py = p[1:2, :]
    pz = p[2:3, :]
    qx = q[:, 0:1]
    qy = q[:, 1:2]
    qz = q[:, 2:3]
    sqq = (qx * qx + qy * qy) + qz * qz
    sqp = (px * px + py * py) + pz * pz
    P = jnp.dot(q, p, preferred_element_type=F32,
                precision=lax.Precision.DEFAULT)
    D = (sqq + sqp) - 2.0 * P
    lane = lax.broadcasted_iota(I32, (NP, N), 1)
    kl = lax.broadcasted_iota(I32, (NP, k), 1)

    def body(j, c):
        D, acc = c
        m = jnp.min(D, 1, keepdims=True)
        sel = jnp.min(jnp.where(D == m, lane, N), 1, keepdims=True)
        acc = jnp.where(kl == j, sel, acc)
        D = jnp.where(lane == sel, jnp.inf, D)
        return D, acc

    _, acc = lax.fori_loop(0, k, body, (D, jnp.zeros((NP, k), I32)))
    if offset:
        acc = acc + pl.program_id(0) * N
    idx_ref[0] = acc


def _knn_all_body(p_ref, q_ref, idx_ref, *, k, nbatch):
    NP = q_ref.shape[1]
    N = p_ref.shape[2]
    rows = nbatch * NP
    Ps, sqqs, sqps = [], [], []
    for b in range(nbatch):
        qb = q_ref[b]
        pb = p_ref[b]
        Ps.append(jnp.dot(qb, pb, preferred_element_type=F32,
                          precision=lax.Precision.DEFAULT))
        qx, qy, qz = qb[:, 0:1], qb[:, 1:2], qb[:, 2:3]
        px, py, pz = pb[0:1, :], pb[1:2, :], pb[2:3, :]
        sqqs.append((qx * qx + qy * qy) + qz * qz)
        sqps.append(jnp.broadcast_to((px * px + py * py) + pz * pz, (NP, N)))
    P = jnp.concatenate(Ps, 0)
    sqq = jnp.concatenate(sqqs, 0)
    sqp = jnp.concatenate(sqps, 0)
    D = (sqq + sqp) - 2.0 * P
    lane = lax.broadcasted_iota(I32, (rows, N), 1)
    kl = lax.broadcasted_iota(I32, (rows, k), 1)

    def body(j, c):
        D, acc = c
        m = jnp.min(D, 1, keepdims=True)
        sel = jnp.min(jnp.where(D == m, lane, N), 1, keepdims=True)
        acc = jnp.where(kl == j, sel, acc)
        D = jnp.where(lane == sel, jnp.inf, D)
        return D, acc

    _, acc = lax.fori_loop(0, k, body, (D, jnp.zeros((rows, k), I32)))
    idx_ref[...] = acc.reshape(nbatch, NP, k)


def _knn_all(p3n, qT, k):
    B, _, N = p3n.shape
    NP = qT.shape[1]
    return pl.pallas_call(
        functools.partial(_knn_all_body, k=k, nbatch=B),
        out_shape=jax.ShapeDtypeStruct((B, NP, k), I32),
    )(p3n, qT)


def _knn(p3n, qT, k, offset=False):
    B, _, N = p3n.shape
    NP = qT.shape[1]
    return pl.pallas_call(
        functools.partial(_knn_body, k=k, offset=offset),
        grid=(B,),
        in_specs=[
            pl.BlockSpec((1, 3, N), lambda i: (i, 0, 0)),
            pl.BlockSpec((1, NP, 3), lambda i: (i, 0, 0)),
        ],
        out_specs=pl.BlockSpec((1, NP, k), lambda i: (i, 0, 0)),
        out_shape=jax.ShapeDtypeStruct((B, NP, k), I32),
    )(p3n, qT)


def _g1post_body(g_ref, q_ref, w_ref, b_ref, y_ref, s_ref, ss_ref, *, ksamp):
    g = g_ref[0]
    R, Dp = g.shape
    q = q_ref[0]
    nq = q.shape[0]
    qb = jnp.broadcast_to(q[:, None, :], (nq, ksamp, 3)).reshape(R, 3)
    x0 = g - jnp.concatenate([qb, jnp.zeros((R, Dp - 3), F32)], axis=1)
    y = jnp.dot(x0, w_ref[...], preferred_element_type=F32,
                precision=lax.Precision.DEFAULT) + b_ref[...]
    y_ref[0] = y

    @pl.when((pl.program_id(0) == 0) & (pl.program_id(1) == 0))
    def _():
        s_ref[...] = jnp.zeros_like(s_ref)
        ss_ref[...] = jnp.zeros_like(ss_ref)

    s_ref[...] += jnp.sum(y, 0, keepdims=True)
    ss_ref[...] += jnp.sum(y * y, 0, keepdims=True)


def _g1post(g, qT, Wp, b, ksamp, rows_per_chunk):
    B, RT, Dp = g.shape
    C = Wp.shape[1]
    nch = RT // rows_per_chunk
    qch = rows_per_chunk // ksamp
    return pl.pallas_call(
        functools.partial(_g1post_body, ksamp=ksamp),
        grid=(B, nch),
        in_specs=[
            pl.BlockSpec((1, rows_per_chunk, Dp), lambda i, j: (i, j, 0)),
            pl.BlockSpec((1, qch, 3), lambda i, j: (i, j, 0)),
            pl.BlockSpec((Dp, C), lambda i, j: (0, 0)),
            pl.BlockSpec((1, C), lambda i, j: (0, 0)),
        ],
        out_specs=(
            pl.BlockSpec((1, rows_per_chunk, C), lambda i, j: (i, j, 0)),
            pl.BlockSpec((1, C), lambda i, j: (0, 0)),
            pl.BlockSpec((1, C), lambda i, j: (0, 0)),
        ),
        out_shape=(
            jax.ShapeDtypeStruct((B, RT, C), F32),
            jax.ShapeDtypeStruct((1, C), F32),
            jax.ShapeDtypeStruct((1, C), F32),
        ),
    )(g, qT, Wp, b)


def _g2_body(idx_ref, tab_ref, q_ref, w_ref, b_ref, y_ref, s_ref, ss_ref, *, ksamp):
    idxc = idx_ref[0]
    R = idxc.shape[0]
    N = tab_ref.shape[1]
    S = (idxc == lax.broadcasted_iota(I32, (R, N), 1)).astype(F32)
    g = jnp.dot(S, tab_ref[0], preferred_element_type=F32, precision=HI)
    Cin = g.shape[1]
    q = q_ref[0]
    nq = q.shape[0]
    qb = jnp.broadcast_to(q[:, None, :], (nq, ksamp, 3)).reshape(R, 3)
    x = g - jnp.concatenate([qb, jnp.zeros((R, Cin - 3), F32)], axis=1)
    y = jnp.dot(x, w_ref[...], preferred_element_type=F32,
                precision=lax.Precision.DEFAULT) + b_ref[...]
    y_ref[0] = y

    @pl.when((pl.program_id(0) == 0) & (pl.program_id(1) == 0))
    def _():
        s_ref[...] = jnp.zeros_like(s_ref)
        ss_ref[...] = jnp.zeros_like(ss_ref)

    s_ref[...] += jnp.sum(y, 0, keepdims=True)
    ss_ref[...] += jnp.sum(y * y, 0, keepdims=True)


def _gather2_mlp(idxf, tab, qT, W, b, ksamp, rows_per_chunk):
    B, RT, _ = idxf.shape
    N, Cin = tab.shape[1], tab.shape[2]
    C = W.shape[1]
    nch = RT // rows_per_chunk
    qch = rows_per_chunk // ksamp
    return pl.pallas_call(
        functools.partial(_g2_body, ksamp=ksamp),
        grid=(B, nch),
        in_specs=[
            pl.BlockSpec((1, rows_per_chunk, 1), lambda i, j: (i, j, 0)),
            pl.BlockSpec((1, N, Cin), lambda i, j: (i, 0, 0)),
            pl.BlockSpec((1, qch, 3), lambda i, j: (i, j, 0)),
            pl.BlockSpec((Cin, C), lambda i, j: (0, 0)),
            pl.BlockSpec((1, C), lambda i, j: (0, 0)),
        ],
        out_specs=(
            pl.BlockSpec((1, rows_per_chunk, C), lambda i, j: (i, j, 0)),
            pl.BlockSpec((1, C), lambda i, j: (0, 0)),
            pl.BlockSpec((1, C), lambda i, j: (0, 0)),
        ),
        out_shape=(
            jax.ShapeDtypeStruct((B, RT, C), F32),
            jax.ShapeDtypeStruct((1, C), F32),
            jax.ShapeDtypeStruct((1, C), F32),
        ),
    )(idxf, tab, qT, W, b)


def _bnmm_body(y_ref, sc_ref, sh_ref, w_ref, b_ref, z_ref, s_ref, ss_ref):
    h = jnp.maximum(y_ref[0] * sc_ref[...] + sh_ref[...], 0.0)
    z = jnp.dot(h, w_ref[...], preferred_element_type=F32,
                precision=lax.Precision.DEFAULT) + b_ref[...]
    z_ref[0] = z

    @pl.when((pl.program_id(0) == 0) & (pl.program_id(1) == 0))
    def _():
        s_ref[...] = jnp.zeros_like(s_ref)
        ss_ref[...] = jnp.zeros_like(ss_ref)

    s_ref[...] += jnp.sum(z, 0, keepdims=True)
    ss_ref[...] += jnp.sum(z * z, 0, keepdims=True)


def _bn_mlp(y, scale, shift, W, b, rows_per_chunk):
    B, RT, Cin = y.shape
    C = W.shape[1]
    nch = RT // rows_per_chunk
    return pl.pallas_call(
        _bnmm_body,
        grid=(B, nch),
        in_specs=[
            pl.BlockSpec((1, rows_per_chunk, Cin), lambda i, j: (i, j, 0)),
            pl.BlockSpec((1, Cin), lambda i, j: (0, 0)),
            pl.BlockSpec((1, Cin), lambda i, j: (0, 0)),
            pl.BlockSpec((Cin, C), lambda i, j: (0, 0)),
            pl.BlockSpec((1, C), lambda i, j: (0, 0)),
        ],
        out_specs=(
            pl.BlockSpec((1, rows_per_chunk, C), lambda i, j: (i, j, 0)),
            pl.BlockSpec((1, C), lambda i, j: (0, 0)),
            pl.BlockSpec((1, C), lambda i, j: (0, 0)),
        ),
        out_shape=(
            jax.ShapeDtypeStruct((B, RT, C), F32),
            jax.ShapeDtypeStruct((1, C), F32),
            jax.ShapeDtypeStruct((1, C), F32),
        ),
    )(y, scale, shift, W, b)


def _mm_body(x_ref, w_ref, b_ref, z_ref, s_ref, ss_ref):
    z = jnp.dot(x_ref[0], w_ref[...], preferred_element_type=F32,
                precision=lax.Precision.DEFAULT) + b_ref[...]
    z_ref[0] = z

    @pl.when(pl.program_id(0) == 0)
    def _():
        s_ref[...] = jnp.zeros_like(s_ref)
        ss_ref[...] = jnp.zeros_like(ss_ref)

    s_ref[...] += jnp.sum(z, 0, keepdims=True)
    ss_ref[...] += jnp.sum(z * z, 0, keepdims=True)


def _mm_stats(x, W, b):
    B, RT, Cin = x.shape
    C = W.shape[1]
    return pl.pallas_call(
        _mm_body,
        grid=(B,),
        in_specs=[
            pl.BlockSpec((1, RT, Cin), lambda i: (i, 0, 0)),
            pl.BlockSpec((Cin, C), lambda i: (0, 0)),
            pl.BlockSpec((1, C), lambda i: (0, 0)),
        ],
        out_specs=(
            pl.BlockSpec((1, RT, C), lambda i: (i, 0, 0)),
            pl.BlockSpec((1, C), lambda i: (0, 0)),
            pl.BlockSpec((1, C), lambda i: (0, 0)),
        ),
        out_shape=(
            jax.ShapeDtypeStruct((B, RT, C), F32),
            jax.ShapeDtypeStruct((1, C), F32),
            jax.ShapeDtypeStruct((1, C), F32),
        ),
    )(x, W, b)


def _maxcat_body(y_ref, sc_ref, sh_ref, q_ref, o_ref, *, ksamp, center):
    h = jnp.maximum(y_ref[0] * sc_ref[...] + sh_ref[...], 0.0)
    R, C = h.shape
    nq = R // ksamp
    p = jnp.max(h.reshape(nq, ksamp, C), axis=1)
    q = q_ref[0]
    if center:
        q = q - jnp.mean(q, axis=0, keepdims=True)
    o_ref[0] = jnp.concatenate([q, p], axis=1)


def _max_cat(y, scale, shift, qT, ksamp, q_chunk, center=False):
    B, RT, C = y.shape
    nq_total = RT // ksamp
    nch = nq_total // q_chunk
    rows = q_chunk * ksamp
    return pl.pallas_call(
        functools.partial(_maxcat_body, ksamp=ksamp, center=center),
        grid=(B, nch),
        in_specs=[
            pl.BlockSpec((1, rows, C), lambda i, j: (i, j, 0)),
            pl.BlockSpec((1, C), lambda i, j: (0, 0)),
            pl.BlockSpec((1, C), lambda i, j: (0, 0)),
            pl.BlockSpec((1, q_chunk, 3), lambda i, j: (i, j, 0)),
        ],
        out_specs=pl.BlockSpec((1, q_chunk, C + 3), lambda i, j: (i, j, 0)),
        out_shape=jax.ShapeDtypeStruct((B, nq_total, C + 3), F32),
    )(y, scale, shift, qT)


def _head_body(y_ref, sc_ref, sh_ref,
               w1_ref, b1_ref, g1_ref, be1_ref,
               w2_ref, b2_ref, g2_ref, be2_ref,
               w3_ref, b3_ref, o_ref):
    h = jnp.maximum(y_ref[...] * sc_ref[...] + sh_ref[...], 0.0)
    x = jnp.max(h, axis=1)

    def fc_bn(x, w, b, g, be):
        a = jnp.dot(x, w[...], preferred_element_type=F32,
                    precision=lax.Precision.DEFAULT) + b[...]
        m = jnp.mean(a, axis=0, keepdims=True)
        v = jnp.mean((a - m) ** 2, axis=0, keepdims=True)
        return jnp.maximum(g[...] * (a - m) / jnp.sqrt(v + EPS) + be[...], 0.0)

    x = fc_bn(x, w1_ref, b1_ref, g1_ref, be1_ref)
    x = fc_bn(x, w2_ref, b2_ref, g2_ref, be2_ref)
    o = jnp.dot(x, w3_ref[...], preferred_element_type=F32,
                precision=lax.Precision.DEFAULT) + b3_ref[...]
    o = o - jnp.max(o, axis=1, keepdims=True)
    o_ref[...] = o - jnp.log(jnp.sum(jnp.exp(o), axis=1, keepdims=True))


def _head(y9, scale, shift, fc1, fc2, fc3):
    B = y9.shape[0]
    nc = fc3['W'].shape[1]
    return pl.pallas_call(
        _head_body,
        out_shape=jax.ShapeDtypeStruct((B, nc), F32),
    )(y9, scale, shift,
      fc1['W'], fc1['b'][None, :], fc1['g'][None, :], fc1['be'][None, :],
      fc2['W'], fc2['b'][None, :], fc2['g'][None, :], fc2['be'][None, :],
      fc3['W'], fc3['b'][None, :])


def _finalize(s, ss, n, g, be):
    mean = s / n
    var = ss / n - mean * mean
    scale = g[None, :] / jnp.sqrt(var + EPS)
    shift = be[None, :] - mean * scale
    return scale, shift


def kernel(xyz, params):
    B, _, N = xyz.shape
    xyzT = jnp.transpose(xyz, (0, 2, 1))

    sa1 = params['sa1']
    q1_3, q1_T = _fps(xyz, 512)
    idx1 = _knn(xyz, q1_T, 32, offset=True)
    n1 = B * 512 * 32
    xyzP = jnp.concatenate([xyzT, jnp.zeros((B, N, 13), F32)], -1).reshape(B * N, 16)
    g1 = _sc_gather(xyzP, idx1.reshape(n1)).reshape(B, 512 * 32, 16)
    L = sa1[0]
    W1p = jnp.concatenate([L['W'], jnp.zeros((13, L['W'].shape[1]), F32)], 0)
    y1, s, ss = _g1post(g1, q1_T, W1p, L['b'][None, :], 32, 4096)
    sc, sh = _finalize(s, ss, n1, sa1[0]['g'], sa1[0]['be'])
    L = sa1[1]
    y2, s, ss = _bn_mlp(y1, sc, sh, L['W'], L['b'][None, :], 8192)
    sc, sh = _finalize(s, ss, n1, L['g'], L['be'])
    L = sa1[2]
    y3, s, ss = _bn_mlp(y2, sc, sh, L['W'], L['b'][None, :], 8192)
    sc, sh = _finalize(s, ss, n1, L['g'], L['be'])
    cat1 = _max_cat(y3, sc, sh, q1_T, 32, 256)

    sa2 = params['sa2']
    q2_3, q2_T = _fps(q1_3, 128)
    idx2 = _knn_all(q1_3, q2_T, 64)
    idx2f = idx2.reshape(B, 128 * 64, 1)
    n2 = B * 128 * 64
    L = sa2[0]
    y4, s, ss = _gather2_mlp(idx2f, cat1, q2_T, L['W'], L['b'][None, :], 64, 2048)
    sc, sh = _finalize(s, ss, n2, L['g'], L['be'])
    L = sa2[1]
    y5, s, ss = _bn_mlp(y4, sc, sh, L['W'], L['b'][None, :], 8192)
    sc, sh = _finalize(s, ss, n2, L['g'], L['be'])
    L = sa2[2]
    y6, s, ss = _bn_mlp(y5, sc, sh, L['W'], L['b'][None, :], 8192)
    sc, sh = _finalize(s, ss, n2, L['g'], L['be'])
    cat2 = _max_cat(y6, sc, sh, q2_T, 64, 128, center=True)

    sa3 = params['sa3']
    n3 = B * 128
    L = sa3[0]
    y7, s, ss = _mm_stats(cat2, L['W'], L['b'][None, :])
    sc, sh = _finalize(s, ss, n3, L['g'], L['be'])
    L = sa3[1]
    y8, s, ss = _bn_mlp(y7, sc, sh, L['W'], L['b'][None, :], 128)
    sc, sh = _finalize(s, ss, n3, L['g'], L['be'])
    L = sa3[2]
    y9, s, ss = _bn_mlp(y8, sc, sh, L['W'], L['b'][None, :], 128)
    sc, sh = _finalize(s, ss, n3, L['g'], L['be'])

    return _head(y9, sc, sh, params['fc1'], params['fc2'], params['fc3'])
